# baseline (speedup 1.0000x reference)
import jax
import jax.numpy as jnp
from jax import lax
from jax.experimental import pallas as pl
from jax.experimental.pallas import tpu as pltpu
from jax.experimental.pallas import tpu_sc as plsc

B = 16
H = 512
W = 512
N_KP = 2048
LANES = 16
BAND = 128
OWN = 256
N_WORKERS = 32


def _sc_body(dens_hbm, kx_hbm, ky_hbm, tg_hbm, out_hbm,
             kx_v, ky_v, tg_v, band_a, acc_v, sem, sem_a):
    dens2 = dens_hbm
    c = lax.axis_index("c")
    s = lax.axis_index("s")
    wid = s * 2 + c
    img = s
    ybase = c * OWN

    cp_b0 = pltpu.async_copy(dens2.at[pl.ds(img * H + ybase, BAND)],
                             band_a, sem_a)
    cp_kx = pltpu.async_copy(kx_hbm.at[img], kx_v, sem)
    cp_ky = pltpu.async_copy(ky_hbm.at[img], ky_v, sem)
    cp_tg = pltpu.async_copy(tg_hbm.at[img], tg_v, sem)
    cp_kx.wait()
    cp_ky.wait()
    cp_tg.wait()

    def _max_step(j, carry):
        mx, my = carry
        xv = kx_v[pl.ds(j * LANES, LANES)]
        yv = ky_v[pl.ds(j * LANES, LANES)]
        return jnp.maximum(mx, xv), jnp.maximum(my, yv)

    mx0 = kx_v[pl.ds(0, LANES)]
    my0 = ky_v[pl.ds(0, LANES)]
    mx, my = lax.fori_loop(1, N_KP // LANES, _max_step, (mx0, my0))
    max_x = jnp.max(mx)
    max_y = jnp.max(my)

    def _scale(mval, dim):
        mvec = lax.broadcast(mval, (LANES,))
        sc = jnp.full((LANES,), jnp.float32(dim)) / mvec
        return jnp.where(mvec > 0, sc, jnp.full((LANES,), jnp.float32(1.0)))

    scale_w = _scale(max_x, W)
    scale_h = _scale(max_y, H)

    acc = jnp.zeros((LANES,), jnp.float32)
    for p in range(2):
        y0 = ybase + p * BAND
        rows = BAND
        band = band_a
        if p == 0:
            cp_b0.wait()
        else:
            pltpu.async_copy(dens2.at[pl.ds(img * H + y0, BAND)], band_a,
                             sem_a).wait()

        def _pass_step(j, acc, y0=y0, rows=rows, band=band):
            xv = kx_v[pl.ds(j * LANES, LANES)]
            yv = ky_v[pl.ds(j * LANES, LANES)]
            tv = tg_v[pl.ds(j * LANES, LANES)]
            ix = jnp.clip((xv * scale_w).astype(jnp.int32), 0, W - 1)
            iy = jnp.clip((yv * scale_h).astype(jnp.int32), 0, H - 1)
            t = iy - y0
            m = (t >= 0) & (t < rows)
            pv = plsc.load_gather(band, [t, ix], mask=m)
            d = jnp.where(m, pv - tv, jnp.float32(0.0))
            return acc + d * d

        acc = lax.fori_loop(0, N_KP // LANES, _pass_step, acc)

    acc_v[...] = acc
    pltpu.async_copy(acc_v, out_hbm.at[wid], sem).wait()


def _tc_finalize_body(part_ref, rgb_ref, th_ref, tot_ref, den_ref, dom_ref):
    alpha = jnp.float32(0.1)
    density_loss = jnp.sum(part_ref[...]) / jnp.float32(B * N_KP)
    lp_rgb = jax.nn.log_softmax(rgb_ref[...], axis=-1)
    lp_th = jax.nn.log_softmax(th_ref[...], axis=-1)
    ce_rgb = -jnp.mean(lp_rgb[:, 0])
    ce_th = -jnp.mean(lp_th[:, 1])
    domain_loss = (ce_rgb + ce_th) * jnp.float32(0.5)
    tot_ref[0] = density_loss + alpha * domain_loss
    den_ref[0] = density_loss
    dom_ref[0] = domain_loss


def kernel(density_map, keypoints_list, targets_list,
           domain_pred_rgb, domain_pred_thermal):
    kx = keypoints_list[:, :, 0]
    ky = keypoints_list[:, :, 1]

    mesh = plsc.VectorSubcoreMesh(core_axis_name="c", subcore_axis_name="s")
    sc_kernel = pl.kernel(
        _sc_body,
        out_type=jax.ShapeDtypeStruct((N_WORKERS, LANES), jnp.float32),
        mesh=mesh,
        scratch_types=[
            pltpu.VMEM((N_KP,), jnp.float32),
            pltpu.VMEM((N_KP,), jnp.float32),
            pltpu.VMEM((N_KP,), jnp.float32),
            pltpu.VMEM((BAND, W), jnp.float32),
            pltpu.VMEM((LANES,), jnp.float32),
            pltpu.SemaphoreType.DMA,
            pltpu.SemaphoreType.DMA,
        ],
        compiler_params=pltpu.CompilerParams(needs_layout_passes=False),
    )
    partials = sc_kernel(density_map.reshape(B * H, W), kx, ky, targets_list)

    tot, den, dom = pl.pallas_call(
        _tc_finalize_body,
        out_shape=[jax.ShapeDtypeStruct((1,), jnp.float32)] * 3,
        out_specs=[pl.BlockSpec(memory_space=pltpu.SMEM)] * 3,
    )(partials, domain_pred_rgb, domain_pred_thermal)

    return (tot.reshape(()), den.reshape(()), dom.reshape(()))
